# Initial kernel scaffold; baseline (speedup 1.0000x reference)
import jax
import jax.numpy as jnp
from jax.experimental import pallas as pl


def kernel(x, edge_index, batch, enc_W, enc_b, c1W, c1b, c2W, c2b, c3W, c3b, g1, be1, g2, be2, g3, be3, eW1, eb1, eW2, eb2, eW3, eb3, kW1, kb1, kW2, kb2, kW3, kb3, gW1, gb1, gW2, gb2, gW3, gb3):
    raise NotImplementedError("write your pallas kernel here")


# speedup vs baseline: 4.7045x; 4.7045x over previous
import functools

import jax
import jax.numpy as jnp
from jax import lax
from jax.experimental import pallas as pl
from jax.experimental.pallas import tpu as pltpu
from jax.experimental.pallas import tpu_sc as plsc

N = 10000
E = 320000
G = 64
H = 256
FS = 64
NF = H // FS
HALF = 5000
PAD = 120
HROWS = HALF + PAD
NP = 2 * HROWS
NC = 2
NS = 16
NW = NC * NS
BLK = 128
NBLK = E // BLK

_mesh = plsc.VectorSubcoreMesh(core_axis_name="c", subcore_axis_name="s")


@functools.partial(
    pl.kernel,
    mesh=_mesh,
    out_type=jax.ShapeDtypeStruct((NC, NP), jnp.float32),
    scratch_types=[
        pltpu.VMEM((BLK,), jnp.int32),
        pltpu.VMEM((BLK,), jnp.float32),
        pltpu.VMEM((NP // NS,), jnp.float32),
        pltpu.VMEM_SHARED((NP,), jnp.float32),
    ],
)
def _deg_kernel(dstp_hbm, out_hbm, idx_v, ones_v, zeros_v, deg_sh):
    c = lax.axis_index("c")
    s = lax.axis_index("s")
    wid = s * NC + c
    zseg = NP // NS
    for i in range(zseg // 16):
        zeros_v[pl.ds(i * 16, 16)] = jnp.zeros((16,), jnp.float32)
    for i in range(BLK // 16):
        ones_v[pl.ds(i * 16, 16)] = jnp.ones((16,), jnp.float32)
    pltpu.sync_copy(zeros_v, deg_sh.at[pl.ds(s * zseg, zseg)])
    plsc.subcore_barrier()

    base_blocks = NBLK // NW
    extra = NBLK - base_blocks * NW
    nblk = jnp.where(wid < extra, base_blocks + 1, base_blocks)

    def body(t, carry):
        e0 = (t * NW + wid) * BLK
        pltpu.sync_copy(dstp_hbm.at[pl.ds(e0, BLK)], idx_v)
        pltpu.sync_copy(ones_v, deg_sh.at[idx_v], add=True)
        return carry

    lax.fori_loop(0, nblk, body, 0)
    plsc.subcore_barrier()
    pltpu.sync_copy(deg_sh.at[pl.ds(s * zseg, zseg)],
                    out_hbm.at[c, pl.ds(s * zseg, zseg)])


_STRIP = jax.ShapeDtypeStruct((NP, FS), jnp.float32)


@functools.partial(
    pl.kernel,
    mesh=_mesh,
    compiler_params=pltpu.CompilerParams(use_tc_tiling_on_sc=False),
    out_type=[_STRIP] * NF,
    scratch_types=(
        [pltpu.VMEM((BLK,), jnp.int32)] * 2
        + [pltpu.VMEM((BLK, FS), jnp.float32)] * NF
        + [pltpu.VMEM((16, FS), jnp.float32)]
        + [pltpu.VMEM_SHARED((HROWS, FS), jnp.float32)] * NF
        + [pltpu.SemaphoreType.DMA]
    ),
)
def _agg_kernel(mp0, mp1, mp2, mp3, srcp_hbm, lds_hbm,
                out0, out1, out2, out3,
                src_v, ldst_v, r0, r1, r2, r3, zrows_v,
                a0, a1, a2, a3, sem):
    c = lax.axis_index("c")
    s = lax.axis_index("s")
    mps = (mp0, mp1, mp2, mp3)
    outs = (out0, out1, out2, out3)
    rows = (r0, r1, r2, r3)
    accs = (a0, a1, a2, a3)
    rpt = HROWS // NS

    for i in range(16):
        for j in range(FS // 16):
            zrows_v[i, pl.ds(j * 16, 16)] = jnp.zeros((16,), jnp.float32)
    for j in range(NF):
        for t in range(rpt // 16):
            pltpu.sync_copy(zrows_v, accs[j].at[pl.ds(s * rpt + t * 16, 16)])
    plsc.subcore_barrier()

    base_blocks = NBLK // NS
    extra = NBLK - base_blocks * NS
    nblk = jnp.where(s < extra, base_blocks + 1, base_blocks)

    def body(t, carry):
        e0 = (t * NS + s) * BLK
        pltpu.sync_copy(srcp_hbm.at[pl.ds(e0, BLK)], src_v)
        pltpu.sync_copy(lds_hbm.at[c, pl.ds(e0, BLK)], ldst_v)
        for j in range(NF):
            pltpu.async_copy(mps[j].at[src_v], rows[j], sem).wait()
            pltpu.sync_copy(rows[j], accs[j].at[ldst_v], add=True)
        return carry

    lax.fori_loop(0, nblk, body, 0)
    plsc.subcore_barrier()
    for j in range(NF):
        pltpu.sync_copy(accs[j].at[pl.ds(s * rpt, rpt)],
                        outs[j].at[pl.ds(c * HROWS + s * rpt, rpt)])


RB = 1024
NB = NP // RB


def _row_valid(base):
    r = base + lax.broadcasted_iota(jnp.int32, (RB, 1), 0)
    return ((r < HALF) | ((r >= HROWS) & (r < HROWS + HALF))).astype(jnp.float32)


def _strip_out(val, refs):
    for j, ref in enumerate(refs):
        ref[...] = val[:, j * FS:(j + 1) * FS]


def _dot3(a, b):
    f32 = jnp.float32
    bf = jnp.bfloat16
    a_hi = a.astype(bf)
    a_lo = (a - a_hi.astype(f32)).astype(bf)
    b_hi = b.astype(bf)
    b_lo = (b - b_hi.astype(f32)).astype(bf)

    def d(p, q):
        return jnp.dot(p, q, preferred_element_type=f32)

    return ((d(a_lo, b_lo) + d(a_hi, b_lo)) + d(a_lo, b_hi)) + d(a_hi, b_hi)


def _tc_enc(x_ref, degt_ref, encW_ref, encb_ref, c1W_ref,
            m0_ref, m1_ref, m2_ref, m3_ref, dinv_ref):
    deg = degt_ref[:, 0:1] + degt_ref[:, 1:2] + 1.0
    dinv = lax.rsqrt(deg)
    dinv_ref[...] = dinv
    h = jnp.maximum(_dot3(x_ref[...], encW_ref[...]) + encb_ref[...], 0.0)
    m1 = dinv * _dot3(h, c1W_ref[...])
    _strip_out(m1, (m0_ref, m1_ref, m2_ref, m3_ref))


def _agg_block(S_refs, mp_refs, dinv, b_ref):
    agg = jnp.concatenate(
        [S_refs[j][...] + mp_refs[j][...] for j in range(NF)], axis=1)
    return dinv * agg + b_ref[...]


def _tc_mid(S0, S1, S2, S3, mp0, mp1, mp2, mp3, dinv_ref, b_ref, g_ref,
            be_ref, Wn_ref, *args, residual):
    if residual:
        res_ref = args[0]
        args = args[1:]
    x_ref, mn0, mn1, mn2, mn3, sum_s, sq_s = args
    p = pl.program_id(0)
    i = pl.program_id(1)
    dinv = dinv_ref[...]
    agg = _agg_block((S0, S1, S2, S3), (mp0, mp1, mp2, mp3), dinv, b_ref)
    valid = _row_valid(i * RB)

    @pl.when((p == 0) & (i == 0))
    def _():
        sum_s[...] = jnp.zeros_like(sum_s)
        sq_s[...] = jnp.zeros_like(sq_s)

    @pl.when(p == 0)
    def _():
        sum_s[...] += jnp.sum(agg * valid, axis=0, keepdims=True)

    @pl.when(p == 1)
    def _():
        mean = sum_s[...] * (1.0 / N)
        d0 = agg - mean
        sq_s[...] += jnp.sum(d0 * d0 * valid, axis=0, keepdims=True)

    @pl.when(p == 2)
    def _():
        mean = sum_s[...] * (1.0 / N)
        var = sq_s[...] * (1.0 / N)
        xn = (agg - mean) * lax.rsqrt(var + 1e-5) * g_ref[...] + be_ref[...]
        x = jnp.maximum(xn, 0.0)
        if residual:
            x = x + res_ref[...]
        x_ref[...] = x
        mn = dinv * _dot3(x, Wn_ref[...])
        _strip_out(mn, (mn0, mn1, mn2, mn3))


def _tc_pool(S0, S1, S2, S3, mp0, mp1, mp2, mp3, dinv_ref, b_ref, g_ref,
             be_ref, res_ref, batch_ref, sums_ref, cnt_ref,
             sum_s, sq_s, psum_s, pcnt_s):
    p = pl.program_id(0)
    i = pl.program_id(1)
    dinv = dinv_ref[...]
    agg = _agg_block((S0, S1, S2, S3), (mp0, mp1, mp2, mp3), dinv, b_ref)
    valid = _row_valid(i * RB)

    @pl.when((p == 0) & (i == 0))
    def _():
        sum_s[...] = jnp.zeros_like(sum_s)
        sq_s[...] = jnp.zeros_like(sq_s)
        psum_s[...] = jnp.zeros_like(psum_s)
        pcnt_s[...] = jnp.zeros_like(pcnt_s)

    @pl.when(p == 0)
    def _():
        sum_s[...] += jnp.sum(agg * valid, axis=0, keepdims=True)

    @pl.when(p == 1)
    def _():
        mean = sum_s[...] * (1.0 / N)
        d0 = agg - mean
        sq_s[...] += jnp.sum(d0 * d0 * valid, axis=0, keepdims=True)

    @pl.when(p == 2)
    def _():
        mean = sum_s[...] * (1.0 / N)
        var = sq_s[...] * (1.0 / N)
        xn = (agg - mean) * lax.rsqrt(var + 1e-5) * g_ref[...] + be_ref[...]
        x3 = jnp.maximum(xn, 0.0) + res_ref[...]
        seg = lax.broadcasted_iota(jnp.int32, (G, RB), 0)
        P = (batch_ref[...] == seg).astype(jnp.float32)
        psum_s[...] += jnp.dot(P, x3, preferred_element_type=jnp.float32, precision=lax.Precision.HIGHEST)
        pcnt_s[...] += jnp.sum(P, axis=1, keepdims=True)
        sums_ref[...] = psum_s[...]
        cnt_ref[...] = pcnt_s[...]


def _tc_heads(sums_ref, cnt_ref,
              eW1_ref, eb1_ref, eW2_ref, eb2_ref, eW3_ref, eb3_ref,
              kW1_ref, kb1_ref, kW2_ref, kb2_ref, kW3_ref, kb3_ref,
              gW1_ref, gb1_ref, gW2_ref, gb2_ref, gW3_ref, gb3_ref,
              e_ref, k_ref, gp_ref):
    pooled = sums_ref[...] / jnp.maximum(cnt_ref[...], 1.0)

    def head(W1, b1, W2, b2, W3, b3):
        hp = lax.Precision.HIGHEST
        h = jnp.tanh(jnp.dot(pooled, W1[...],
                             preferred_element_type=jnp.float32, precision=hp) + b1[...])
        h = jnp.tanh(jnp.dot(h, W2[...],
                             preferred_element_type=jnp.float32, precision=hp) + b2[...])
        return jnp.dot(h, W3[...], preferred_element_type=jnp.float32,
                       precision=hp) + b3[...]

    e_ref[...] = head(eW1_ref, eb1_ref, eW2_ref, eb2_ref, eW3_ref, eb3_ref)
    k_ref[...] = head(kW1_ref, kb1_ref, kW2_ref, kb2_ref, kW3_ref, kb3_ref)
    gp_ref[...] = head(gW1_ref, gb1_ref, gW2_ref, gb2_ref, gW3_ref, gb3_ref)


def _f32(shape):
    return jax.ShapeDtypeStruct(shape, jnp.float32)


def kernel(x, edge_index, batch, enc_W, enc_b, c1W, c1b, c2W, c2b, c3W, c3b,
           g1, be1, g2, be2, g3, be3,
           eW1, eb1, eW2, eb2, eW3, eb3,
           kW1, kb1, kW2, kb2, kW3, kb3,
           gW1, gb1, gW2, gb2, gW3, gb3):
    f32 = jnp.float32
    i32 = jnp.int32
    src = edge_index[0]
    dst = edge_index[1]
    srcp = src + jnp.where(src >= HALF, PAD, 0).astype(i32)
    dstp = dst + jnp.where(dst >= HALF, PAD, 0).astype(i32)
    garb = HALF + (jnp.arange(E, dtype=i32) % PAD)
    in0 = dst < HALF
    lds = jnp.stack([jnp.where(in0, dst, garb),
                     jnp.where(in0, garb, dst - HALF)])

    zpad = jnp.zeros((PAD, x.shape[1]), f32)
    xp = jnp.concatenate([x[:HALF], zpad, x[HALF:], zpad], axis=0)
    bpad = jnp.full((PAD,), -1, i32)
    batch_row = jnp.concatenate(
        [batch[:HALF], bpad, batch[HALF:], bpad])[None, :]

    degp = _deg_kernel(dstp)
    degt = degp.T

    row = lambda v: v[None, :]
    strip4 = [_f32((NP, FS))] * NF

    def rb(c):
        return pl.BlockSpec((RB, c), lambda p, i: (i, 0))

    def full(r, c):
        return pl.BlockSpec((r, c), lambda p, i: (0, 0))

    stat_scratch = [pltpu.VMEM((1, H), f32), pltpu.VMEM((1, H), f32)]

    rb1 = lambda c: pl.BlockSpec((RB, c), lambda i: (i, 0))
    *m1s, dinv = pl.pallas_call(
        _tc_enc,
        grid=(NB,),
        in_specs=[rb1(x.shape[1]), rb1(2),
                  pl.BlockSpec((x.shape[1], H), lambda i: (0, 0)),
                  pl.BlockSpec((1, H), lambda i: (0, 0)),
                  pl.BlockSpec((H, H), lambda i: (0, 0))],
        out_specs=[rb1(FS)] * NF + [rb1(1)],
        out_shape=strip4 + [_f32((NP, 1))],
    )(xp, degt, enc_W, row(enc_b), c1W)

    def mid(Ss, mps, b, g, be, Wn, res):
        specs = ([rb(FS)] * 8
                 + [rb(1), full(1, H), full(1, H), full(1, H), full(H, H)])
        argv = [*Ss, *mps, dinv, row(b), row(g), row(be), Wn]
        if res is not None:
            specs.append(rb(H))
            argv.append(res)
        return pl.pallas_call(
            functools.partial(_tc_mid, residual=res is not None),
            grid=(2, NB),
            in_specs=specs,
            out_specs=[rb(H)] + [rb(FS)] * NF,
            out_shape=[_f32((NP, H))] + strip4,
            scratch_shapes=stat_scratch,
        )(*argv)

    S1 = _agg_kernel(*m1s, srcp, lds)
    x1, *m2s = mid(S1, m1s, c1b, g1, be1, c2W, None)

    S2 = _agg_kernel(*m2s, srcp, lds)
    x2, *m3s = mid(S2, m2s, c2b, g2, be2, c3W, x1)

    S3 = _agg_kernel(*m3s, srcp, lds)
    sums, cnt = pl.pallas_call(
        _tc_pool,
        grid=(3, NB),
        in_specs=([rb(FS)] * 8
                  + [rb(1), full(1, H), full(1, H), full(1, H), rb(H),
                     pl.BlockSpec((1, RB), lambda p, i: (0, i))]),
        out_specs=[full(G, H), full(G, 1)],
        out_shape=[_f32((G, H)), _f32((G, 1))],
        scratch_shapes=stat_scratch + [pltpu.VMEM((G, H), f32),
                                       pltpu.VMEM((G, 1), f32)],
    )(*S3, *m3s, dinv, row(c3b), row(g3), row(be3), x2, batch_row)

    energy, ks_gap, e_gap = pl.pallas_call(
        _tc_heads,
        out_shape=[_f32((G, 1)), _f32((G, 1)), _f32((G, 1))],
    )(sums, cnt,
      eW1, row(eb1), eW2, row(eb2), eW3, row(eb3),
      kW1, row(kb1), kW2, row(kb2), kW3, row(kb3),
      gW1, row(gb1), gW2, row(gb2), gW3, row(gb3))
    return (energy, ks_gap, e_gap)
